# Initial kernel scaffold; baseline (speedup 1.0000x reference)
import jax
import jax.numpy as jnp
from jax.experimental import pallas as pl


def kernel(user_idx, item_idx, hist_idx, user_table, item_table, hist_table):
    raise NotImplementedError("write your pallas kernel here")


# speedup vs baseline: 1.6054x; 1.6054x over previous
import functools

import jax
import jax.numpy as jnp
from jax import lax
from jax.experimental import pallas as pl
from jax.experimental.pallas import tpu as pltpu
from jax.experimental.pallas import tpu_sc as plsc

B = 4096
L = 50
D = 32
LANES = 16


def _embed_kernel_body(u_idx, i_idx, h_idx, u_tab, i_tab, h_tab,
                       u_out, i_out, h_out,
                       uidx_v, iidx_v, hidx_v, urows, irows, acc,
                       sem_idx, sem_ui, sem_h):
    nc = lax.axis_index("c")
    ns = lax.axis_index("s")
    wid = ns * 2 + nc
    bpw = B // 32
    base = wid * bpw

    cp_u = pltpu.async_copy(u_idx.at[pl.ds(base, bpw)], uidx_v, sem_idx)
    cp_i = pltpu.async_copy(i_idx.at[pl.ds(base, bpw)], iidx_v, sem_idx)
    cp_h = pltpu.async_copy(h_idx.at[wid], hidx_v, sem_idx)

    zeros = jnp.zeros((LANES,), jnp.float32)

    def zbody(b, carry):
        acc[b, pl.ds(0, LANES)] = zeros
        acc[b, pl.ds(LANES, LANES)] = zeros
        return carry

    lax.fori_loop(0, bpw, zbody, 0)

    cp_u.wait()
    cp_i.wait()
    cp_h.wait()

    g_u = pltpu.async_copy(u_tab.at[uidx_v], urows, sem_ui)
    g_i = pltpu.async_copy(i_tab.at[iidx_v], irows, sem_ui)

    def fire(p, carry):
        pltpu.async_copy(h_tab.at[hidx_v.at[p]], acc, sem_h, add=True)
        return carry

    lax.fori_loop(0, L, fire, 0)

    g_u.wait()
    g_i.wait()
    st_u = pltpu.async_copy(urows, u_out.at[pl.ds(base, bpw)], sem_ui)
    st_i = pltpu.async_copy(irows, i_out.at[pl.ds(base, bpw)], sem_ui)

    def drain(p, carry):
        pltpu.make_async_copy(h_tab.at[hidx_v.at[0]], acc, sem_h).wait()
        return carry

    lax.fori_loop(0, L, drain, 0)

    scale = jnp.full((LANES,), 1.0 / L, jnp.float32)

    def sbody(b, carry):
        acc[b, pl.ds(0, LANES)] = acc[b, pl.ds(0, LANES)] * scale
        acc[b, pl.ds(LANES, LANES)] = acc[b, pl.ds(LANES, LANES)] * scale
        return carry

    lax.fori_loop(0, bpw, sbody, 0)

    pltpu.sync_copy(acc, h_out.at[pl.ds(base, bpw)])
    st_u.wait()
    st_i.wait()


@jax.jit
def kernel(user_idx, item_idx, hist_idx, user_table, item_table, hist_table):
    bpw = B // 32
    u_idx = user_idx.reshape(B).astype(jnp.int32)
    i_idx = item_idx.reshape(B).astype(jnp.int32)
    h_idx = (hist_idx.astype(jnp.int32)
             .reshape(32, bpw, L)
             .transpose(0, 2, 1))

    mesh = plsc.VectorSubcoreMesh(core_axis_name="c", subcore_axis_name="s")
    run = functools.partial(
        pl.kernel,
        out_type=[
            jax.ShapeDtypeStruct((B, D), jnp.float32),
            jax.ShapeDtypeStruct((B, D), jnp.float32),
            jax.ShapeDtypeStruct((B, D), jnp.float32),
        ],
        mesh=mesh,
        compiler_params=pltpu.CompilerParams(use_tc_tiling_on_sc=False),
        scratch_types=[
            pltpu.VMEM((bpw,), jnp.int32),
            pltpu.VMEM((bpw,), jnp.int32),
            pltpu.VMEM((L, bpw), jnp.int32),
            pltpu.VMEM((bpw, D), jnp.float32),
            pltpu.VMEM((bpw, D), jnp.float32),
            pltpu.VMEM((bpw, D), jnp.float32),
            pltpu.SemaphoreType.DMA,
            pltpu.SemaphoreType.DMA,
            pltpu.SemaphoreType.DMA,
        ],
    )(_embed_kernel_body)

    u_rows, i_rows, h_rows = run(u_idx, i_idx, h_idx,
                                 user_table, item_table, hist_table)
    return jnp.stack([u_rows, i_rows, h_rows], axis=1)
